# Initial kernel scaffold; baseline (speedup 1.0000x reference)
import dataclasses
import functools

import jax
import jax.numpy as jnp
from jax import lax
from jax.experimental import pallas as pl
from jax.experimental.pallas import tpu as pltpu
from jax.experimental.pallas import tpu_sc as plsc

NC = 2
NS = 16
NW = NC * NS
L = 16
UNIT = 128
ROW = 8

_CP = pltpu.CompilerParams()
if "needs_layout_passes" in pltpu.CompilerParams.__dataclass_fields__:
    _CP = dataclasses.replace(_CP, needs_layout_passes=False)

_MESH = plsc.VectorSubcoreMesh(core_axis_name="c", subcore_axis_name="s")


def _iota():
    return lax.iota(jnp.int32, L)


def _cvec(c):
    return jnp.full((L,), c, jnp.int32)


def _fvec(x):
    return jnp.full((L,), x, jnp.float32)


def _rsqrt(x):
    i = plsc.bitcast(x, jnp.int32)
    i = jnp.full((L,), 0x5F3759DF, jnp.int32) - lax.shift_right_logical(
        i, jnp.full((L,), 1, jnp.int32))
    y = plsc.bitcast(i, jnp.float32)
    h = x * _fvec(0.5)
    for _ in range(3):
        y = y * (_fvec(1.5) - h * y * y)
    return y


def _accumulate_kernel(V, Vp, Fp):
    FW = Fp // NW
    NUNITS = FW // UNIT
    ZR = Vp // NS

    @functools.partial(
        pl.kernel,
        mesh=_MESH,
        out_type=jax.ShapeDtypeStruct((NC, Vp, ROW), jnp.float32),
        scratch_types=[
            pltpu.VMEM_SHARED((Vp, ROW), jnp.float32),
            pltpu.VMEM((UNIT,), jnp.int32),
            pltpu.VMEM((UNIT,), jnp.int32),
            pltpu.VMEM((UNIT,), jnp.int32),
            pltpu.VMEM((UNIT,), jnp.int32),
            pltpu.VMEM((UNIT,), jnp.int32),
            pltpu.VMEM((UNIT,), jnp.int32),
            pltpu.VMEM((UNIT, ROW), jnp.float32),
            pltpu.VMEM((UNIT, ROW), jnp.float32),
            pltpu.VMEM((UNIT, ROW), jnp.float32),
            pltpu.VMEM((UNIT, ROW), jnp.float32),
            pltpu.VMEM((UNIT, ROW), jnp.float32),
            pltpu.VMEM((UNIT, ROW), jnp.float32),
            pltpu.VMEM((UNIT, ROW), jnp.float32),
        ],
        compiler_params=_CP,
    )
    def k(table_h, fidx_h, zero_h, out_h, acc,
          i0, i1, i2, j0, j1, j2, p0, p1, p2, t0, t1, t2, res):
        cid = lax.axis_index("c")
        sid = lax.axis_index("s")
        wid = cid * NS + sid
        pltpu.sync_copy(zero_h, acc.at[pl.ds(sid * ZR, ZR), :])
        pltpu.sync_copy(zero_h.at[pl.ds(0, UNIT), :], res)
        plsc.subcore_barrier()

        base_w = wid * FW
        iota = _iota()

        @pl.loop(0, NUNITS)
        def _(u):
            b = base_w + u * UNIT
            pltpu.sync_copy(fidx_h.at[0, pl.ds(b, UNIT)], i0)
            pltpu.sync_copy(fidx_h.at[1, pl.ds(b, UNIT)], i1)
            pltpu.sync_copy(fidx_h.at[2, pl.ds(b, UNIT)], i2)
            pltpu.sync_copy(fidx_h.at[3, pl.ds(b, UNIT)], j0)
            pltpu.sync_copy(fidx_h.at[4, pl.ds(b, UNIT)], j1)
            pltpu.sync_copy(fidx_h.at[5, pl.ds(b, UNIT)], j2)
            pltpu.sync_copy(table_h.at[i0], p0)
            pltpu.sync_copy(table_h.at[i1], p1)
            pltpu.sync_copy(table_h.at[i2], p2)
            pltpu.sync_copy(table_h.at[j0], t0)
            pltpu.sync_copy(table_h.at[j1], t1)
            pltpu.sync_copy(table_h.at[j2], t2)
            for g in range(UNIT // L):
                rows = iota + _cvec(g * L)

                def ld(ref, c):
                    return plsc.load_gather(ref, [rows, _cvec(c)])

                p0x, p0y, p0z = ld(p0, 0), ld(p0, 1), ld(p0, 2)
                p1x, p1y, p1z = ld(p1, 0), ld(p1, 1), ld(p1, 2)
                p2x, p2y, p2z = ld(p2, 0), ld(p2, 1), ld(p2, 2)
                t0u, t0v = ld(t0, 0), ld(t0, 1)
                t1u, t1v = ld(t1, 0), ld(t1, 1)
                t2u, t2v = ld(t2, 0), ld(t2, 1)
                e1x, e1y, e1z = p1x - p0x, p1y - p0y, p1z - p0z
                e2x, e2y, e2z = p2x - p0x, p2y - p0y, p2z - p0z
                nx = e1y * e2z - e1z * e2y
                ny = e1z * e2x - e1x * e2z
                nz = e1x * e2y - e1y * e2x
                u1, v1 = t1u - t0u, t1v - t0v
                u2, v2 = t2u - t0u, t2v - t0v
                den = u1 * v2 - v1 * u2
                den_safe = jnp.where(den > _fvec(0.0),
                                     jnp.maximum(den, _fvec(1e-6)),
                                     jnp.minimum(den, _fvec(-1e-6)))
                inv = _fvec(1.0) / den_safe
                tgx = (e1x * v2 - e2x * v1) * inv
                tgy = (e1y * v2 - e2y * v1) * inv
                tgz = (e1z * v2 - e2z * v1) * inv
                plsc.store_scatter(res, [rows, _cvec(0)], nx)
                plsc.store_scatter(res, [rows, _cvec(1)], ny)
                plsc.store_scatter(res, [rows, _cvec(2)], nz)
                plsc.store_scatter(res, [rows, _cvec(3)], tgx)
                plsc.store_scatter(res, [rows, _cvec(4)], tgy)
                plsc.store_scatter(res, [rows, _cvec(5)], tgz)
            pltpu.sync_copy(res, acc.at[i0], add=True)
            pltpu.sync_copy(res, acc.at[i1], add=True)
            pltpu.sync_copy(res, acc.at[i2], add=True)

        plsc.subcore_barrier()
        pltpu.sync_copy(acc.at[pl.ds(sid * ZR, ZR), :],
                        out_h.at[cid, pl.ds(sid * ZR, ZR), :])

    return k


def _finalize_kernel(Vp):
    WV = Vp // NW

    @functools.partial(
        pl.kernel,
        mesh=_MESH,
        out_type=jax.ShapeDtypeStruct((2, Vp, 4), jnp.float32),
        scratch_types=[
            pltpu.VMEM((WV, ROW), jnp.float32),
            pltpu.VMEM((WV, ROW), jnp.float32),
            pltpu.VMEM((WV, 4), jnp.float32),
            pltpu.VMEM((WV, 4), jnp.float32),
        ],
        compiler_params=_CP,
    )
    def k(in_h, out_h, a0, a1, nout, tout):
        cid = lax.axis_index("c")
        sid = lax.axis_index("s")
        wid = cid * NS + sid
        b = wid * WV
        pltpu.sync_copy(in_h.at[0, pl.ds(b, WV), :], a0)
        pltpu.sync_copy(in_h.at[1, pl.ds(b, WV), :], a1)
        iota = _iota()

        @pl.loop(0, WV // L)
        def _(g):
            rows = iota + g * L

            def ld(c):
                cc = _cvec(c)
                return (plsc.load_gather(a0, [rows, cc]) +
                        plsc.load_gather(a1, [rows, cc]))

            nx, ny, nz = ld(0), ld(1), ld(2)
            tx, ty, tz = ld(3), ld(4), ld(5)
            d = nx * nx + ny * ny + nz * nz
            cond = d > _fvec(1e-20)
            zero = _fvec(0.0)
            nx = jnp.where(cond, nx, zero)
            ny = jnp.where(cond, ny, zero)
            nz = jnp.where(cond, nz, _fvec(1.0))
            dsel = jnp.where(cond, d, _fvec(1.0))
            r = _rsqrt(jnp.maximum(dsel, _fvec(1e-20)))
            onx, ony, onz = nx * r, ny * r, nz * r
            dt = tx * tx + ty * ty + tz * tz
            rt = _rsqrt(jnp.maximum(dt, _fvec(1e-20)))
            ttx, tty, ttz = tx * rt, ty * rt, tz * rt
            dtn = ttx * onx + tty * ony + ttz * onz
            wx = ttx - dtn * onx
            wy = tty - dtn * ony
            wz = ttz - dtn * onz
            dw = wx * wx + wy * wy + wz * wz
            rw = _rsqrt(jnp.maximum(dw, _fvec(1e-20)))
            plsc.store_scatter(nout, [rows, _cvec(0)], onx)
            plsc.store_scatter(nout, [rows, _cvec(1)], ony)
            plsc.store_scatter(nout, [rows, _cvec(2)], onz)
            plsc.store_scatter(tout, [rows, _cvec(0)], wx * rw)
            plsc.store_scatter(tout, [rows, _cvec(1)], wy * rw)
            plsc.store_scatter(tout, [rows, _cvec(2)], wz * rw)

        pltpu.sync_copy(nout, out_h.at[0, pl.ds(b, WV), :])
        pltpu.sync_copy(tout, out_h.at[1, pl.ds(b, WV), :])

    return k


def kernel(positions, texcoords, faces, uv_faces):
    V = positions.shape[0]
    F = faces.shape[0]
    per_w = -(-F // (NW * UNIT)) * UNIT
    Fp = per_w * NW
    Vp = -(-V // (NW * L)) * (NW * L)

    table = jnp.zeros((2 * V, ROW), jnp.float32)
    table = table.at[:V, :3].set(positions.astype(jnp.float32))
    table = table.at[V:, :2].set(texcoords.astype(jnp.float32))
    f_t = faces.astype(jnp.int32).T
    u_t = uv_faces.astype(jnp.int32).T + V
    fidx = jnp.concatenate([f_t, u_t], axis=0)
    fidx = jnp.pad(fidx, ((0, 0), (0, Fp - F)))
    zero = jnp.zeros((Vp // NS, ROW), jnp.float32)

    partial = _accumulate_kernel(V, Vp, Fp)(table, fidx, zero)
    out2 = _finalize_kernel(Vp)(partial)
    return jnp.concatenate([out2[0, :V, :3], out2[1, :V, :3]], axis=0)


# speedup vs baseline: 5.6808x; 5.6808x over previous
import dataclasses
import functools

import jax
import jax.numpy as jnp
from jax import lax
from jax.experimental import pallas as pl
from jax.experimental.pallas import tpu as pltpu
from jax.experimental.pallas import tpu_sc as plsc

NC = 2
NS = 16
NW = NC * NS
L = 16
UNIT = 128
ROW = 8

_CP = pltpu.CompilerParams(use_tc_tiling_on_sc=False)
if "needs_layout_passes" in pltpu.CompilerParams.__dataclass_fields__:
    _CP = dataclasses.replace(_CP, needs_layout_passes=False)

_MESH = plsc.VectorSubcoreMesh(core_axis_name="c", subcore_axis_name="s")


def _iota():
    return lax.iota(jnp.int32, L)


def _cvec(c):
    return jnp.full((L,), c, jnp.int32)


def _fvec(x):
    return jnp.full((L,), x, jnp.float32)


def _rsqrt(x):
    i = plsc.bitcast(x, jnp.int32)
    i = jnp.full((L,), 0x5F3759DF, jnp.int32) - lax.shift_right_logical(
        i, jnp.full((L,), 1, jnp.int32))
    y = plsc.bitcast(i, jnp.float32)
    h = x * _fvec(0.5)
    for _ in range(3):
        y = y * (_fvec(1.5) - h * y * y)
    return y


def _accumulate_kernel(V, Vp, Fp):
    FW = Fp // NW
    NUNITS = FW // UNIT
    ZR = Vp // NS

    @functools.partial(
        pl.kernel,
        mesh=_MESH,
        out_type=jax.ShapeDtypeStruct((NC, Vp, ROW), jnp.float32),
        scratch_types=[
            pltpu.VMEM_SHARED((Vp, ROW), jnp.float32),
            pltpu.VMEM((UNIT,), jnp.int32),
            pltpu.VMEM((UNIT,), jnp.int32),
            pltpu.VMEM((UNIT,), jnp.int32),
            pltpu.VMEM((UNIT,), jnp.int32),
            pltpu.VMEM((UNIT,), jnp.int32),
            pltpu.VMEM((UNIT,), jnp.int32),
            pltpu.VMEM((UNIT, ROW), jnp.float32),
            pltpu.VMEM((UNIT, ROW), jnp.float32),
            pltpu.VMEM((UNIT, ROW), jnp.float32),
            pltpu.VMEM((UNIT, ROW), jnp.float32),
            pltpu.VMEM((UNIT, ROW), jnp.float32),
            pltpu.VMEM((UNIT, ROW), jnp.float32),
            pltpu.VMEM((UNIT, ROW), jnp.float32),
        ],
        compiler_params=_CP,
    )
    def k(table_h, fidx_h, zero_h, out_h, acc,
          i0, i1, i2, j0, j1, j2, p0, p1, p2, t0, t1, t2, res):
        cid = lax.axis_index("c")
        sid = lax.axis_index("s")
        wid = cid * NS + sid
        pltpu.sync_copy(zero_h, acc.at[pl.ds(sid * ZR, ZR), :])
        pltpu.sync_copy(zero_h.at[pl.ds(0, UNIT), :], res)
        plsc.subcore_barrier()

        base_w = wid * FW
        iota = _iota()

        @pl.loop(0, NUNITS)
        def _(u):
            b = base_w + u * UNIT
            pltpu.sync_copy(fidx_h.at[0, pl.ds(b, UNIT)], i0)
            pltpu.sync_copy(fidx_h.at[1, pl.ds(b, UNIT)], i1)
            pltpu.sync_copy(fidx_h.at[2, pl.ds(b, UNIT)], i2)
            pltpu.sync_copy(fidx_h.at[3, pl.ds(b, UNIT)], j0)
            pltpu.sync_copy(fidx_h.at[4, pl.ds(b, UNIT)], j1)
            pltpu.sync_copy(fidx_h.at[5, pl.ds(b, UNIT)], j2)
            pltpu.sync_copy(table_h.at[i0], p0)
            pltpu.sync_copy(table_h.at[i1], p1)
            pltpu.sync_copy(table_h.at[i2], p2)
            pltpu.sync_copy(table_h.at[j0], t0)
            pltpu.sync_copy(table_h.at[j1], t1)
            pltpu.sync_copy(table_h.at[j2], t2)
            for g in range(UNIT // L):
                rows = iota + _cvec(g * L)

                def ld(ref, c):
                    return plsc.load_gather(ref, [rows, _cvec(c)])

                p0x, p0y, p0z = ld(p0, 0), ld(p0, 1), ld(p0, 2)
                p1x, p1y, p1z = ld(p1, 0), ld(p1, 1), ld(p1, 2)
                p2x, p2y, p2z = ld(p2, 0), ld(p2, 1), ld(p2, 2)
                t0u, t0v = ld(t0, 0), ld(t0, 1)
                t1u, t1v = ld(t1, 0), ld(t1, 1)
                t2u, t2v = ld(t2, 0), ld(t2, 1)
                e1x, e1y, e1z = p1x - p0x, p1y - p0y, p1z - p0z
                e2x, e2y, e2z = p2x - p0x, p2y - p0y, p2z - p0z
                nx = e1y * e2z - e1z * e2y
                ny = e1z * e2x - e1x * e2z
                nz = e1x * e2y - e1y * e2x
                u1, v1 = t1u - t0u, t1v - t0v
                u2, v2 = t2u - t0u, t2v - t0v
                den = u1 * v2 - v1 * u2
                den_safe = jnp.where(den > _fvec(0.0),
                                     jnp.maximum(den, _fvec(1e-6)),
                                     jnp.minimum(den, _fvec(-1e-6)))
                inv = _fvec(1.0) / den_safe
                tgx = (e1x * v2 - e2x * v1) * inv
                tgy = (e1y * v2 - e2y * v1) * inv
                tgz = (e1z * v2 - e2z * v1) * inv
                plsc.store_scatter(res, [rows, _cvec(0)], nx)
                plsc.store_scatter(res, [rows, _cvec(1)], ny)
                plsc.store_scatter(res, [rows, _cvec(2)], nz)
                plsc.store_scatter(res, [rows, _cvec(3)], tgx)
                plsc.store_scatter(res, [rows, _cvec(4)], tgy)
                plsc.store_scatter(res, [rows, _cvec(5)], tgz)
            pltpu.sync_copy(res, acc.at[i0], add=True)
            pltpu.sync_copy(res, acc.at[i1], add=True)
            pltpu.sync_copy(res, acc.at[i2], add=True)

        plsc.subcore_barrier()
        pltpu.sync_copy(acc.at[pl.ds(sid * ZR, ZR), :],
                        out_h.at[cid, pl.ds(sid * ZR, ZR), :])

    return k


def _finalize_kernel(Vp):
    WV = Vp // NW

    @functools.partial(
        pl.kernel,
        mesh=_MESH,
        out_type=jax.ShapeDtypeStruct((2, Vp, 4), jnp.float32),
        scratch_types=[
            pltpu.VMEM((WV, ROW), jnp.float32),
            pltpu.VMEM((WV, ROW), jnp.float32),
            pltpu.VMEM((WV, 4), jnp.float32),
            pltpu.VMEM((WV, 4), jnp.float32),
        ],
        compiler_params=_CP,
    )
    def k(in_h, out_h, a0, a1, nout, tout):
        cid = lax.axis_index("c")
        sid = lax.axis_index("s")
        wid = cid * NS + sid
        b = wid * WV
        pltpu.sync_copy(in_h.at[0, pl.ds(b, WV), :], a0)
        pltpu.sync_copy(in_h.at[1, pl.ds(b, WV), :], a1)
        iota = _iota()

        @pl.loop(0, WV // L)
        def _(g):
            rows = iota + g * L

            def ld(c):
                cc = _cvec(c)
                return (plsc.load_gather(a0, [rows, cc]) +
                        plsc.load_gather(a1, [rows, cc]))

            nx, ny, nz = ld(0), ld(1), ld(2)
            tx, ty, tz = ld(3), ld(4), ld(5)
            d = nx * nx + ny * ny + nz * nz
            cond = d > _fvec(1e-20)
            zero = _fvec(0.0)
            nx = jnp.where(cond, nx, zero)
            ny = jnp.where(cond, ny, zero)
            nz = jnp.where(cond, nz, _fvec(1.0))
            dsel = jnp.where(cond, d, _fvec(1.0))
            r = _rsqrt(jnp.maximum(dsel, _fvec(1e-20)))
            onx, ony, onz = nx * r, ny * r, nz * r
            dt = tx * tx + ty * ty + tz * tz
            rt = _rsqrt(jnp.maximum(dt, _fvec(1e-20)))
            ttx, tty, ttz = tx * rt, ty * rt, tz * rt
            dtn = ttx * onx + tty * ony + ttz * onz
            wx = ttx - dtn * onx
            wy = tty - dtn * ony
            wz = ttz - dtn * onz
            dw = wx * wx + wy * wy + wz * wz
            rw = _rsqrt(jnp.maximum(dw, _fvec(1e-20)))
            plsc.store_scatter(nout, [rows, _cvec(0)], onx)
            plsc.store_scatter(nout, [rows, _cvec(1)], ony)
            plsc.store_scatter(nout, [rows, _cvec(2)], onz)
            plsc.store_scatter(tout, [rows, _cvec(0)], wx * rw)
            plsc.store_scatter(tout, [rows, _cvec(1)], wy * rw)
            plsc.store_scatter(tout, [rows, _cvec(2)], wz * rw)

        pltpu.sync_copy(nout, out_h.at[0, pl.ds(b, WV), :])
        pltpu.sync_copy(tout, out_h.at[1, pl.ds(b, WV), :])

    return k


def kernel(positions, texcoords, faces, uv_faces):
    V = positions.shape[0]
    F = faces.shape[0]
    per_w = -(-F // (NW * UNIT)) * UNIT
    Fp = per_w * NW
    Vp = -(-V // (NW * L)) * (NW * L)

    table = jnp.zeros((2 * V, ROW), jnp.float32)
    table = table.at[:V, :3].set(positions.astype(jnp.float32))
    table = table.at[V:, :2].set(texcoords.astype(jnp.float32))
    f_t = faces.astype(jnp.int32).T
    u_t = uv_faces.astype(jnp.int32).T + V
    fidx = jnp.concatenate([f_t, u_t], axis=0)
    fidx = jnp.pad(fidx, ((0, 0), (0, Fp - F)))
    zero = jnp.zeros((Vp // NS, ROW), jnp.float32)

    partial = _accumulate_kernel(V, Vp, Fp)(table, fidx, zero)
    out2 = _finalize_kernel(Vp)(partial)
    return jnp.concatenate([out2[0, :V, :3], out2[1, :V, :3]], axis=0)
